# baseline (speedup 1.0000x reference)
import functools

import jax
import jax.numpy as jnp
from jax import lax
from jax.experimental import pallas as pl
from jax.experimental.pallas import tpu as pltpu
from jax.experimental.pallas import tpu_sc as plsc

NC = 2
NS = 16
L = 16
NW = NC * NS

B = 16384
D = 64
USER_N = 100000
ITEM_N = 1000000
U_BANDS = (USER_N + 127) // 128
I_BANDS = (ITEM_N + 127) // 128
U_PB = -(-U_BANDS // NW)
I_PB = -(-I_BANDS // NW)
U_PAD = U_BANDS * 128
CHUNK = 4096
HMAX = 4096
WIN = 3
NWIN = -(-I_PB // WIN)
PMAX = 128
NGR = PMAX // L
PAD = 2 * B

_CP = pltpu.CompilerParams(needs_layout_passes=False,
                           use_tc_tiling_on_sc=True)


def _detile_user_body(ut_hbm, ustage_hbm, bb, st, sem, sem2):
    wid = lax.axis_index("s") * NC + lax.axis_index("c")
    lo = wid * U_PB
    hi = jnp.minimum(lo + U_PB, U_BANDS)
    lane = lax.iota(jnp.int32, L)
    n = U_PB

    def fetch(i, s):
        c = jnp.minimum(lo + i, hi - 1)
        off = pl.multiple_of(c * 128, 128)
        pltpu.async_copy(ut_hbm.at[:, pl.ds(off, 128)], bb.at[s], sem)

    fetch(0, 0)

    def band_body(i, _):
        s = i & 1

        @pl.when(i + 1 < n)
        def _():
            fetch(i + 1, 1 - s)

        pltpu.make_async_copy(ut_hbm.at[:, pl.ds(0, 128)], bb.at[s],
                              sem).wait()
        for d in range(D):
            col = jnp.full((L,), d, jnp.int32)
            sv = jnp.full((L,), s, jnp.int32)
            for p in range(8):
                v = bb[s, d, pl.ds(p * L, L)]
                plsc.store_scatter(st, [sv, p * L + lane, col], v)

        @pl.when(i > 0)
        def _():
            pltpu.make_async_copy(st.at[1 - s],
                                  ustage_hbm.at[pl.ds(0, 128), :],
                                  sem2).wait()

        c = jnp.minimum(lo + i, hi - 1)
        off = pl.multiple_of(c * 128, 128)
        pltpu.async_copy(st.at[s], ustage_hbm.at[pl.ds(off, 128), :], sem2)
        return 0

    lax.fori_loop(0, n, band_body, 0)
    pltpu.make_async_copy(st.at[(n - 1) & 1],
                          ustage_hbm.at[pl.ds(0, 128), :], sem2).wait()


def _item_body(user_hbm, item_i_hbm, item_j_hbm, it_hbm, ustage_hbm,
               pred_hbm,
               uid_all, ichunk, hb, hidx, pb, pcol, uidb, bb, urow, res,
               sem, sem2):
    wid = lax.axis_index("s") * NC + lax.axis_index("c")
    lo = wid * I_PB
    hi = jnp.minimum(lo + I_PB, I_BANDS)
    lane = lax.iota(jnp.int32, L)

    pltpu.sync_copy(user_hbm, uid_all)

    def scan_stream(src_hbm, boff, ptr0):
        def chunk_body(k, ptr):
            pltpu.sync_copy(src_hbm.at[pl.ds(k * CHUNK, CHUNK)], ichunk)

            def vec_body(q, ptr):
                iv = ichunk[pl.ds(q * L, L)]
                band = iv >> 7
                m = (band >= lo) & (band < hi)
                bv = boff + k * CHUNK + q * L + lane
                pc = jnp.minimum(ptr, HMAX)
                plsc.store_compressed(hb.at[pl.ds(pc, L)], bv, mask=m)
                plsc.store_compressed(hidx.at[pl.ds(pc, L)], iv, mask=m)
                return ptr + plsc.all_reduce_population_count(m)[0]

            return lax.fori_loop(0, CHUNK // L, vec_body, ptr)

        return lax.fori_loop(0, B // CHUNK, chunk_body, ptr0)

    ptr = scan_stream(item_i_hbm, 0, jnp.int32(0))
    ptr = scan_stream(item_j_hbm, B, ptr)
    ptr = jnp.minimum(ptr, HMAX)
    nvec = (ptr + L - 1) // L

    def rescan(w, s):
        c0 = lo + w * WIN
        cend = jnp.minimum(c0 + WIN, hi)
        padv = jnp.full((L,), PAD, jnp.int32)
        for g in range(NGR):
            pb[s, pl.ds(g * L, L)] = padv
            pcol[s, pl.ds(g * L, L)] = jnp.zeros((L,), jnp.int32)

        def pack_body(q, p2):
            ok = (q * L + lane) < ptr
            iv = hidx[pl.ds(q * L, L)]
            bv = hb[pl.ds(q * L, L)]
            band = iv >> 7
            m = ok & (band >= c0) & (band < cend)
            scol = (band - c0) * 128 + (iv & 127)
            p2c = jnp.minimum(p2, PMAX - L)
            plsc.store_compressed(pb.at[s, pl.ds(p2c, L)], bv, mask=m)
            plsc.store_compressed(pcol.at[s, pl.ds(p2c, L)], scol, mask=m)
            return p2 + plsc.all_reduce_population_count(m)[0]

        p2 = jnp.minimum(lax.fori_loop(0, nvec, pack_body, jnp.int32(0)),
                         PMAX)
        for g in range(NGR):
            ev = pb[s, pl.ds(g * L, L)]
            uid = plsc.load_gather(uid_all, [ev & (B - 1)])
            uidb[s, pl.ds(g * L, L)] = uid
        return p2

    def fire(w, s, p2):
        for k in range(WIN):
            c = jnp.minimum(lo + w * WIN + k, hi - 1)
            off = pl.multiple_of(c * 128, 128)
            pltpu.async_copy(it_hbm.at[:, pl.ds(off, 128)],
                             bb.at[s, :, pl.ds(k * 128, 128)], sem)
        ngrp = (p2 + L - 1) // L
        for g in range(NGR):
            @pl.when(g < ngrp)
            def _():
                pltpu.async_copy(
                    ustage_hbm.at[uidb.at[s, pl.ds(g * L, L)]],
                    urow.at[s, pl.ds(g * L, L), :], sem)

    p2_0 = rescan(0, 0)
    fire(0, 0, p2_0)

    def window_body(w, carry):
        p2w, p2prev = carry
        s = w & 1
        s2 = 1 - s

        ngpp = (p2prev + L - 1) // L
        for g in range(NGR):
            @pl.when(g < ngpp)
            def _():
                pltpu.make_async_copy(res.at[s2, pl.ds(g * L, L)],
                                      pred_hbm.at[pl.ds(0, L)], sem2).wait()

        p2n = rescan(w + 1, s2)

        @pl.when(w + 1 < NWIN)
        def _():
            fire(w + 1, s2, p2n)

        for k in range(WIN):
            pltpu.make_async_copy(it_hbm.at[:, pl.ds(0, 128)],
                                  bb.at[s, :, pl.ds(k * 128, 128)],
                                  sem).wait()
        ngw = (p2w + L - 1) // L
        for g in range(NGR):
            @pl.when(g < ngw)
            def _():
                pltpu.make_async_copy(ustage_hbm.at[pl.ds(0, L), :],
                                      urow.at[s, pl.ds(g * L, L), :],
                                      sem).wait()

        sv = jnp.full((L,), s, jnp.int32)
        for g in range(NGR):
            @pl.when(g < ngw)
            def _():
                scol = pcol[s, pl.ds(g * L, L)]
                acc = jnp.zeros((L,), jnp.float32)
                for d in range(D):
                    dv = jnp.full((L,), d, jnp.int32)
                    iv_d = plsc.load_gather(bb, [sv, dv, scol])
                    u_d = plsc.load_gather(urow, [sv, g * L + lane, dv])
                    acc = acc + iv_d * u_d
                res[s, pl.ds(g * L, L)] = acc

        for g in range(NGR):
            @pl.when(g < ngw)
            def _():
                pltpu.async_copy(res.at[s, pl.ds(g * L, L)],
                                 pred_hbm.at[pb.at[s, pl.ds(g * L, L)]],
                                 sem2)

        return (p2n, p2w)

    _, p2last = lax.fori_loop(0, NWIN, window_body, (p2_0, jnp.int32(0)))
    nglast = (p2last + L - 1) // L
    for g in range(NGR):
        @pl.when(g < nglast)
        def _():
            pltpu.make_async_copy(res.at[(NWIN - 1) & 1, pl.ds(g * L, L)],
                                  pred_hbm.at[pl.ds(0, L)], sem2).wait()


@jax.jit
def _bpr(user, item_i, item_j, embed_user_weight, embed_item_weight):
    mesh = plsc.VectorSubcoreMesh(core_axis_name="c", subcore_axis_name="s",
                                  num_cores=NC, num_subcores=NS)
    k1 = functools.partial(
        pl.kernel,
        out_type=jax.ShapeDtypeStruct((U_PAD, 128), jnp.float32),
        mesh=mesh,
        compiler_params=_CP,
        scratch_types=[
            pltpu.VMEM((2, D, 128), jnp.float32),
            pltpu.VMEM((2, 128, 128), jnp.float32),
            pltpu.SemaphoreType.DMA,
            pltpu.SemaphoreType.DMA,
        ],
    )(_detile_user_body)
    ustage = k1(embed_user_weight.T)

    k2 = functools.partial(
        pl.kernel,
        out_type=jax.ShapeDtypeStruct((PAD + L,), jnp.float32),
        mesh=mesh,
        compiler_params=_CP,
        scratch_types=[
            pltpu.VMEM((B,), jnp.int32),
            pltpu.VMEM((CHUNK,), jnp.int32),
            pltpu.VMEM((HMAX + L,), jnp.int32),
            pltpu.VMEM((HMAX + L,), jnp.int32),
            pltpu.VMEM((2, PMAX), jnp.int32),
            pltpu.VMEM((2, PMAX), jnp.int32),
            pltpu.VMEM((2, PMAX), jnp.int32),
            pltpu.VMEM((2, D, WIN * 128), jnp.float32),
            pltpu.VMEM((2, PMAX, 128), jnp.float32),
            pltpu.VMEM((2, PMAX), jnp.float32),
            pltpu.SemaphoreType.DMA,
            pltpu.SemaphoreType.DMA,
        ],
    )(_item_body)
    pred = k2(user, item_i, item_j, embed_item_weight.T, ustage)
    return pred[:B], pred[B:PAD]


def kernel(user, item_i, item_j, embed_user_weight, embed_item_weight):
    return _bpr(user, item_i, item_j, embed_user_weight, embed_item_weight)
